# Initial kernel scaffold; baseline (speedup 1.0000x reference)
import jax
import jax.numpy as jnp
from jax.experimental import pallas as pl


def kernel(x, edge_index, W1, b1, W2, b2, W3, b3):
    raise NotImplementedError("write your pallas kernel here")


# speedup vs baseline: 39.0599x; 39.0599x over previous
import functools

import jax
import jax.numpy as jnp
from jax import lax
from jax.experimental import pallas as pl
from jax.experimental.pallas import tpu as pltpu
from jax.experimental.pallas import tpu_sc as plsc

_N = 10000
_E = 320000
_D_IN = 128
_H = 64
_NC = 2
_NS = 16
_NW = _NC * _NS
_EPW = _E // _NW
_B = 400
_NCH = _EPW // _B
_BD = 2000
_NCHD = _EPW // _BD

_mesh = plsc.VectorSubcoreMesh(
    core_axis_name="c", subcore_axis_name="s", num_cores=_NC, num_subcores=_NS
)
_sc_params = pltpu.CompilerParams(use_tc_tiling_on_sc=False)

_f32 = jnp.float32


@functools.partial(
    pl.kernel,
    out_type=jax.ShapeDtypeStruct((_NC, _N), _f32),
    mesh=_mesh,
    compiler_params=_sc_params,
    scratch_types=[
        pltpu.VMEM((_BD,), jnp.int32),
        pltpu.VMEM((_BD,), _f32),
        pltpu.VMEM_SHARED((_N,), _f32),
    ],
)
def _deg_kernel(dst_hbm, ones_hbm, zeros_hbm, out_hbm, idx_v, ones_v, acc):
    c = lax.axis_index("c")
    s = lax.axis_index("s")
    w = c * _NS + s

    @pl.when(s == 0)
    def _():
        pltpu.sync_copy(zeros_hbm, acc)

    pltpu.sync_copy(ones_hbm, ones_v)
    plsc.subcore_barrier()
    base = w * _EPW
    for k in range(_NCHD):
        pltpu.sync_copy(dst_hbm.at[pl.ds(base + k * _BD, _BD)], idx_v)
        pltpu.sync_copy(ones_v, acc.at[idx_v], add=True)
    plsc.subcore_barrier()

    @pl.when(s == 0)
    def _():
        pltpu.sync_copy(acc, out_hbm.at[c])


@functools.partial(
    pl.kernel,
    out_type=jax.ShapeDtypeStruct((_NC, _N, _H), _f32),
    mesh=_mesh,
    compiler_params=_sc_params,
    scratch_types=[
        pltpu.VMEM((_B,), jnp.int32),
        pltpu.VMEM((_B,), jnp.int32),
        pltpu.VMEM((_B,), jnp.int32),
        pltpu.VMEM((_B,), jnp.int32),
        pltpu.VMEM((2, _B, _H), _f32),
        pltpu.VMEM_SHARED((_N, _H), _f32),
        pltpu.SemaphoreType.DMA,
        pltpu.SemaphoreType.DMA,
        pltpu.SemaphoreType.DMA,
        pltpu.SemaphoreType.DMA,
    ],
)
def _agg_kernel(h_hbm, src_hbm, dst_hbm, zeros_hbm, out_hbm,
                sidx0, sidx1, didx0, didx1, rows, acc,
                gsem0, gsem1, ssem0, ssem1):
    c = lax.axis_index("c")
    s = lax.axis_index("s")
    w = c * _NS + s

    @pl.when(s == 0)
    def _():
        pltpu.sync_copy(zeros_hbm, acc)

    plsc.subcore_barrier()

    gsems = [gsem0, gsem1]
    ssems = [ssem0, ssem1]
    sidx = [sidx0, sidx1]
    didx = [didx0, didx1]
    gd = [None, None]
    sd = [None, None]
    base = w * _EPW

    pltpu.sync_copy(src_hbm.at[pl.ds(base, _B)], sidx[0])
    pltpu.sync_copy(dst_hbm.at[pl.ds(base, _B)], didx[0])
    gd[0] = pltpu.async_copy(h_hbm.at[sidx[0]], rows.at[0], gsems[0])

    for k in range(_NCH):
        b = k % 2
        nb = (k + 1) % 2
        if k + 1 < _NCH:
            if k >= 1:
                sd[nb].wait()
            off = base + (k + 1) * _B
            pltpu.sync_copy(src_hbm.at[pl.ds(off, _B)], sidx[nb])
            pltpu.sync_copy(dst_hbm.at[pl.ds(off, _B)], didx[nb])
            gd[nb] = pltpu.async_copy(h_hbm.at[sidx[nb]], rows.at[nb], gsems[nb])
        gd[b].wait()
        sd[b] = pltpu.async_copy(rows.at[b], acc.at[didx[b]], ssems[b], add=True)

    if _NCH >= 2:
        sd[(_NCH - 2) % 2].wait()
    sd[(_NCH - 1) % 2].wait()
    plsc.subcore_barrier()

    rpt = 632

    @pl.when(s < _NS - 1)
    def _():
        pltpu.sync_copy(acc.at[pl.ds(s * rpt, rpt)],
                        out_hbm.at[c, pl.ds(s * rpt, rpt)])

    @pl.when(s == _NS - 1)
    def _():
        last = _N - (_NS - 1) * rpt
        pltpu.sync_copy(acc.at[pl.ds((_NS - 1) * rpt, last)],
                        out_hbm.at[c, pl.ds((_NS - 1) * rpt, last)])


@functools.partial(
    pl.kernel,
    out_type=jax.ShapeDtypeStruct((_NC, _N), _f32),
    mesh=_mesh,
    compiler_params=_sc_params,
    scratch_types=[
        pltpu.VMEM((_BD,), jnp.int32),
        pltpu.VMEM((_BD,), jnp.int32),
        pltpu.VMEM((_BD,), _f32),
        pltpu.VMEM_SHARED((_N,), _f32),
        pltpu.SemaphoreType.DMA,
    ],
)
def _agg1_kernel(h_hbm, src_hbm, dst_hbm, zeros_hbm, out_hbm,
                 sidx, didx, vals, acc, gsem):
    c = lax.axis_index("c")
    s = lax.axis_index("s")
    w = c * _NS + s

    @pl.when(s == 0)
    def _():
        pltpu.sync_copy(zeros_hbm, acc)

    plsc.subcore_barrier()
    base = w * _EPW
    for k in range(_NCHD):
        off = base + k * _BD
        pltpu.sync_copy(src_hbm.at[pl.ds(off, _BD)], sidx)
        pltpu.sync_copy(dst_hbm.at[pl.ds(off, _BD)], didx)
        pltpu.async_copy(h_hbm.at[sidx], vals, gsem).wait()
        pltpu.sync_copy(vals, acc.at[didx], add=True)
    plsc.subcore_barrier()

    @pl.when(s == 0)
    def _():
        pltpu.sync_copy(acc, out_hbm.at[c])


def _tc1_body(degp_ref, x_ref, w_ref, dinv_ref, hp_ref):
    deg = degp_ref[0] + degp_ref[1] + 1.0
    dinv = lax.rsqrt(deg)
    dinv_ref[...] = dinv
    h = jnp.dot(x_ref[...], w_ref[...], preferred_element_type=_f32)
    hp_ref[...] = h * dinv


def _tc_mid_body(aggp_ref, hp_ref, dinv_ref, b_ref, w_ref, out_ref):
    dinv = dinv_ref[...]
    pre = (aggp_ref[0] + aggp_ref[1] + hp_ref[...]) * dinv + b_ref[...]
    t = jnp.maximum(pre, 0.0)
    out_ref[...] = jnp.dot(t, w_ref[...], preferred_element_type=_f32) * dinv


def _tc_last_body(aggp_ref, hp_ref, dinv_ref, b_ref, out_ref):
    out_ref[...] = (aggp_ref[0] + aggp_ref[1] + hp_ref[...]) * dinv_ref[...] + b_ref[...]


_tc1 = pl.pallas_call(
    _tc1_body,
    out_shape=(
        jax.ShapeDtypeStruct((_N, 1), _f32),
        jax.ShapeDtypeStruct((_N, _H), _f32),
    ),
)

_tc2 = pl.pallas_call(
    _tc_mid_body, out_shape=jax.ShapeDtypeStruct((_N, _H), _f32)
)

_tc3 = pl.pallas_call(
    _tc_mid_body, out_shape=jax.ShapeDtypeStruct((_N, 1), _f32)
)

_tc4 = pl.pallas_call(
    _tc_last_body, out_shape=jax.ShapeDtypeStruct((_N, 1), _f32)
)


def kernel(x, edge_index, W1, b1, W2, b2, W3, b3):
    src = edge_index[0]
    dst = edge_index[1]
    ones_bd = jnp.ones((_BD,), _f32)
    zeros_n = jnp.zeros((_N,), _f32)
    zeros_nh = jnp.zeros((_N, _H), _f32)

    degp = _deg_kernel(dst, ones_bd, zeros_n)
    dinv, h1p = _tc1(degp.reshape(_NC, _N, 1), x, W1)
    agg1 = _agg_kernel(h1p, src, dst, zeros_nh)
    h2p = _tc2(agg1, h1p, dinv, b1, W2)
    agg2 = _agg_kernel(h2p, src, dst, zeros_nh)
    h3p = _tc3(agg2, h2p, dinv, b2, W3)
    agg3 = _agg1_kernel(h3p.reshape(_N), src, dst, zeros_n)
    out = _tc4(agg3.reshape(_NC, _N, 1), h3p, dinv, b3)
    return out
